# baseline (speedup 1.0000x reference)
import functools

import jax
import jax.numpy as jnp
from jax.experimental import pallas as pl
from jax.experimental.pallas import tpu as pltpu


def _pick_block(n, pref):
    b = min(pref, n)
    while n % b or b % 8:
        b -= 8 if b > 8 else 1
        if b < 8:
            return n
    return b


def _bf(v):
    return v.astype(jnp.bfloat16)


def _k1_body(x_ref, xs_ref, w_ref, b_ref, o_ref):
    w = _bf(w_ref[...])
    b = b_ref[0:1, :]
    h1 = jnp.dot(_bf(x_ref[...]), w, preferred_element_type=jnp.float32) + b
    h2 = jnp.dot(_bf(xs_ref[...]), w, preferred_element_type=jnp.float32) + b
    o_ref[...] = _bf(jnp.concatenate([h1, h2], axis=1))


def _k2_body(a_ref, h_ref, w1_ref, w2_ref, t_ref, cs_ref, acc_ref, *,
             n_rel, inv_rel):
    i = pl.program_id(0)
    r = pl.program_id(1)
    t = jax.nn.relu(jnp.dot(_bf(a_ref[0]), h_ref[...],
                            preferred_element_type=jnp.float32))

    @pl.when(r == 0)
    def _init():
        acc_ref[...] = t

    @pl.when(r != 0)
    def _acc():
        acc_ref[...] += t

    @pl.when(r == n_rel - 1)
    def _fin():
        hid_bf = _bf(acc_ref[...] * inv_rel)
        zh = jax.nn.relu(jnp.dot(hid_bf, _bf(w1_ref[...]),
                                 preferred_element_type=jnp.float32))
        y = jnp.dot(_bf(zh), _bf(w2_ref[...]),
                    preferred_element_type=jnp.float32)
        t_ref[...] = _bf(y)

        @pl.when(i == 0)
        def _zero():
            cs_ref[...] = jnp.zeros_like(cs_ref)

        cs_ref[0:1, :] += jnp.sum(hid_bf.astype(jnp.float32), axis=0,
                                  keepdims=True)


def _k4_body(a_ref, t_ref, beta_ref, z_ref, cs_ref, *, d):
    i = pl.program_id(0)
    a0 = _bf(a_ref[0]).astype(jnp.float32)
    a1 = _bf(a_ref[1]).astype(jnp.float32)
    bp0 = beta_ref[0, 0]
    bp1 = beta_ref[0, 1]
    bn0 = beta_ref[1, 0]
    bn1 = beta_ref[1, 1]
    na_p = _bf(bp0 * a0 + bp1 * a1)
    na_n = _bf(bn0 * a0 + bn1 * a1)
    zp = jax.nn.relu(jnp.dot(na_p, t_ref[:, :d],
                             preferred_element_type=jnp.float32))
    zn = jax.nn.relu(jnp.dot(na_n, t_ref[:, d:],
                             preferred_element_type=jnp.float32))
    z = jnp.concatenate([zp, zn], axis=1)
    z_ref[...] = _bf(z)

    @pl.when(i == 0)
    def _zero():
        cs_ref[...] = jnp.zeros_like(cs_ref)

    cs_ref[0:1, :] += jnp.sum(z, axis=0, keepdims=True)


def _k5_body(z_ref, w_ref, o_ref, *, d):
    zf = z_ref[...].astype(jnp.float32)
    wf = w_ref[...].astype(jnp.float32)
    prod = zf * wf
    scp = jnp.sum(prod[:, :d], axis=1, keepdims=True)
    scn = jnp.sum(prod[:, d:], axis=1, keepdims=True)
    o_ref[...] = jnp.concatenate([scp, scn], axis=1)


def kernel(fts, adjs_norm, fts_shuf, W_pre, b_pre, a_rel, W_hier, W_hgcn, W_disc):
    n_rel, n, _ = adjs_norm.shape
    f = fts.shape[-1]
    d = W_pre.shape[-1]
    d2 = 2 * d

    x = fts[0]
    xs = fts_shuf[0]
    b2 = b_pre.reshape(1, d)

    rb1 = _pick_block(n, 2000)
    hcat = pl.pallas_call(
        _k1_body,
        grid=(n // rb1,),
        in_specs=[
            pl.BlockSpec((rb1, f), lambda i: (i, 0)),
            pl.BlockSpec((rb1, f), lambda i: (i, 0)),
            pl.BlockSpec((f, d), lambda i: (0, 0)),
            pl.BlockSpec((1, d), lambda i: (0, 0)),
        ],
        out_specs=pl.BlockSpec((rb1, d2), lambda i: (i, 0)),
        out_shape=jax.ShapeDtypeStruct((n, d2), jnp.bfloat16),
    )(x, xs, W_pre, b2)

    rb = _pick_block(n, 200)
    nr_g = n // rb

    zblk = jnp.zeros((d, d), jnp.float32)
    w1bd = jnp.block([[W_hier, zblk], [zblk, W_hier]])
    w2bd = jnp.block([[W_hgcn, zblk], [zblk, W_hgcn]])

    tcat, cs_hid = pl.pallas_call(
        functools.partial(_k2_body, n_rel=n_rel, inv_rel=1.0 / n_rel),
        grid=(nr_g, n_rel),
        in_specs=[
            pl.BlockSpec((1, rb, n), lambda i, r: (r, i, 0)),
            pl.BlockSpec((n, d2), lambda i, r: (0, 0)),
            pl.BlockSpec((d2, d2), lambda i, r: (0, 0)),
            pl.BlockSpec((d2, d2), lambda i, r: (0, 0)),
        ],
        out_specs=[
            pl.BlockSpec((rb, d2), lambda i, r: (i, 0)),
            pl.BlockSpec((8, d2), lambda i, r: (0, 0)),
        ],
        out_shape=[
            jax.ShapeDtypeStruct((n, d2), jnp.bfloat16),
            jax.ShapeDtypeStruct((8, d2), jnp.float32),
        ],
        scratch_shapes=[pltpu.VMEM((rb, d2), jnp.float32)],
    )(adjs_norm, hcat, w1bd, w2bd)

    arel_bf = a_rel.astype(jnp.bfloat16).astype(jnp.float32)
    cs = cs_hid[0]
    scores_p = arel_bf @ cs[:d] / n
    scores_n = arel_bf @ cs[d:] / n
    beta_p = jax.nn.softmax(scores_p).astype(jnp.bfloat16).astype(jnp.float32)
    beta_n = jax.nn.softmax(scores_n).astype(jnp.bfloat16).astype(jnp.float32)
    beta2 = jnp.zeros((8, 128), jnp.float32)
    beta2 = beta2.at[0, :n_rel].set(beta_p)
    beta2 = beta2.at[1, :n_rel].set(beta_n)

    rb4 = _pick_block(n, 80)
    zcat, cs_z = pl.pallas_call(
        functools.partial(_k4_body, d=d),
        grid=(n // rb4,),
        in_specs=[
            pl.BlockSpec((n_rel, rb4, n), lambda i: (0, i, 0)),
            pl.BlockSpec((n, d2), lambda i: (0, 0)),
            pl.BlockSpec((8, 128), lambda i: (0, 0)),
        ],
        out_specs=[
            pl.BlockSpec((rb4, d2), lambda i: (i, 0)),
            pl.BlockSpec((8, d2), lambda i: (0, 0)),
        ],
        out_shape=[
            jax.ShapeDtypeStruct((n, d2), jnp.bfloat16),
            jax.ShapeDtypeStruct((8, d2), jnp.float32),
        ],
    )(adjs_norm, tcat, beta2)

    s = jax.nn.sigmoid(cs_z[0, :d] / n)
    s_bf = s.astype(jnp.bfloat16).astype(jnp.float32)
    wdisc_bf = W_disc.astype(jnp.bfloat16).astype(jnp.float32)
    w = (wdisc_bf @ s_bf).astype(jnp.bfloat16)
    w2 = jnp.concatenate([w, w]).reshape(1, d2)

    sc = pl.pallas_call(
        functools.partial(_k5_body, d=d),
        grid=(n // rb1,),
        in_specs=[
            pl.BlockSpec((rb1, d2), lambda i: (i, 0)),
            pl.BlockSpec((1, d2), lambda i: (0, 0)),
        ],
        out_specs=pl.BlockSpec((rb1, 2), lambda i: (i, 0)),
        out_shape=jax.ShapeDtypeStruct((n, 2), jnp.float32),
    )(zcat, w2)

    logits = jnp.concatenate([sc[:, 0], sc[:, 1]]).reshape(1, 2 * n)
    return logits


# speedup vs baseline: 1.1196x; 1.1196x over previous
import functools

import jax
import jax.numpy as jnp
from jax.experimental import pallas as pl
from jax.experimental.pallas import tpu as pltpu


def _pick_block(n, pref):
    b = min(pref, n)
    while n % b or b % 8:
        b -= 8 if b > 8 else 1
        if b < 8:
            return n
    return b


def _bf(v):
    return v.astype(jnp.bfloat16)


def _k1_body(x_ref, xs_ref, w_ref, b_ref, o_ref):
    w = _bf(w_ref[...])
    b = b_ref[0:1, :]
    h1 = jnp.dot(_bf(x_ref[...]), w, preferred_element_type=jnp.float32) + b
    h2 = jnp.dot(_bf(xs_ref[...]), w, preferred_element_type=jnp.float32) + b
    o_ref[...] = _bf(jnp.concatenate([h1, h2], axis=1))


def _k2_body(a_ref, h_ref, w1_ref, w2_ref, t_ref, cs_ref, acc_ref, *,
             n_rel, inv_rel):
    i = pl.program_id(0)
    r = pl.program_id(1)
    t = jax.nn.relu(jnp.dot(_bf(a_ref[0]), h_ref[...],
                            preferred_element_type=jnp.float32))

    @pl.when(r == 0)
    def _init():
        acc_ref[...] = t

    @pl.when(r != 0)
    def _acc():
        acc_ref[...] += t

    @pl.when(r == n_rel - 1)
    def _fin():
        hid_bf = _bf(acc_ref[...] * inv_rel)
        zh = jax.nn.relu(jnp.dot(hid_bf, _bf(w1_ref[...]),
                                 preferred_element_type=jnp.float32))
        y = jnp.dot(_bf(zh), _bf(w2_ref[...]),
                    preferred_element_type=jnp.float32)
        t_ref[...] = _bf(y)

        @pl.when(i == 0)
        def _zero():
            cs_ref[...] = jnp.zeros_like(cs_ref)

        cs_ref[0:1, :] += jnp.sum(hid_bf.astype(jnp.float32), axis=0,
                                  keepdims=True)


def _k4_body(a_ref, t_ref, beta_ref, z_ref, cs_ref, *, d):
    i = pl.program_id(0)
    a0 = _bf(a_ref[0]).astype(jnp.float32)
    a1 = _bf(a_ref[1]).astype(jnp.float32)
    bp0 = beta_ref[0, 0]
    bp1 = beta_ref[0, 1]
    bn0 = beta_ref[1, 0]
    bn1 = beta_ref[1, 1]
    na_p = _bf(bp0 * a0 + bp1 * a1)
    na_n = _bf(bn0 * a0 + bn1 * a1)
    zp = jax.nn.relu(jnp.dot(na_p, t_ref[:, :d],
                             preferred_element_type=jnp.float32))
    zn = jax.nn.relu(jnp.dot(na_n, t_ref[:, d:],
                             preferred_element_type=jnp.float32))
    z = jnp.concatenate([zp, zn], axis=1)
    z_ref[...] = _bf(z)

    @pl.when(i == 0)
    def _zero():
        cs_ref[...] = jnp.zeros_like(cs_ref)

    cs_ref[0:1, :] += jnp.sum(z, axis=0, keepdims=True)


def _k5_body(z_ref, w_ref, o_ref, *, d):
    zf = z_ref[...].astype(jnp.float32)
    wf = w_ref[...].astype(jnp.float32)
    prod = zf * wf
    scp = jnp.sum(prod[:, :d], axis=1, keepdims=True)
    scn = jnp.sum(prod[:, d:], axis=1, keepdims=True)
    o_ref[...] = jnp.concatenate([scp, scn], axis=1)


def kernel(fts, adjs_norm, fts_shuf, W_pre, b_pre, a_rel, W_hier, W_hgcn, W_disc):
    n_rel, n, _ = adjs_norm.shape
    f = fts.shape[-1]
    d = W_pre.shape[-1]
    d2 = 2 * d

    x = fts[0]
    xs = fts_shuf[0]
    b2 = b_pre.reshape(1, d)

    rb1 = _pick_block(n, 2000)
    hcat = pl.pallas_call(
        _k1_body,
        grid=(n // rb1,),
        in_specs=[
            pl.BlockSpec((rb1, f), lambda i: (i, 0)),
            pl.BlockSpec((rb1, f), lambda i: (i, 0)),
            pl.BlockSpec((f, d), lambda i: (0, 0)),
            pl.BlockSpec((1, d), lambda i: (0, 0)),
        ],
        out_specs=pl.BlockSpec((rb1, d2), lambda i: (i, 0)),
        out_shape=jax.ShapeDtypeStruct((n, d2), jnp.bfloat16),
    )(x, xs, W_pre, b2)

    rb = _pick_block(n, 400)
    nr_g = n // rb

    zblk = jnp.zeros((d, d), jnp.float32)
    w1bd = jnp.block([[W_hier, zblk], [zblk, W_hier]])
    w2bd = jnp.block([[W_hgcn, zblk], [zblk, W_hgcn]])

    tcat, cs_hid = pl.pallas_call(
        functools.partial(_k2_body, n_rel=n_rel, inv_rel=1.0 / n_rel),
        grid=(nr_g, n_rel),
        in_specs=[
            pl.BlockSpec((1, rb, n), lambda i, r: (r, i, 0)),
            pl.BlockSpec((n, d2), lambda i, r: (0, 0)),
            pl.BlockSpec((d2, d2), lambda i, r: (0, 0)),
            pl.BlockSpec((d2, d2), lambda i, r: (0, 0)),
        ],
        out_specs=[
            pl.BlockSpec((rb, d2), lambda i, r: (i, 0)),
            pl.BlockSpec((8, d2), lambda i, r: (0, 0)),
        ],
        out_shape=[
            jax.ShapeDtypeStruct((n, d2), jnp.bfloat16),
            jax.ShapeDtypeStruct((8, d2), jnp.float32),
        ],
        scratch_shapes=[pltpu.VMEM((rb, d2), jnp.float32)],
    )(adjs_norm, hcat, w1bd, w2bd)

    arel_bf = a_rel.astype(jnp.bfloat16).astype(jnp.float32)
    cs = cs_hid[0]
    scores_p = arel_bf @ cs[:d] / n
    scores_n = arel_bf @ cs[d:] / n
    beta_p = jax.nn.softmax(scores_p).astype(jnp.bfloat16).astype(jnp.float32)
    beta_n = jax.nn.softmax(scores_n).astype(jnp.bfloat16).astype(jnp.float32)
    beta2 = jnp.zeros((8, 128), jnp.float32)
    beta2 = beta2.at[0, :n_rel].set(beta_p)
    beta2 = beta2.at[1, :n_rel].set(beta_n)

    rb4 = _pick_block(n, 200)
    zcat, cs_z = pl.pallas_call(
        functools.partial(_k4_body, d=d),
        grid=(n // rb4,),
        in_specs=[
            pl.BlockSpec((n_rel, rb4, n), lambda i: (0, i, 0)),
            pl.BlockSpec((n, d2), lambda i: (0, 0)),
            pl.BlockSpec((8, 128), lambda i: (0, 0)),
        ],
        out_specs=[
            pl.BlockSpec((rb4, d2), lambda i: (i, 0)),
            pl.BlockSpec((8, d2), lambda i: (0, 0)),
        ],
        out_shape=[
            jax.ShapeDtypeStruct((n, d2), jnp.bfloat16),
            jax.ShapeDtypeStruct((8, d2), jnp.float32),
        ],
    )(adjs_norm, tcat, beta2)

    s = jax.nn.sigmoid(cs_z[0, :d] / n)
    s_bf = s.astype(jnp.bfloat16).astype(jnp.float32)
    wdisc_bf = W_disc.astype(jnp.bfloat16).astype(jnp.float32)
    w = (wdisc_bf @ s_bf).astype(jnp.bfloat16)
    w2 = jnp.concatenate([w, w]).reshape(1, d2)

    sc = pl.pallas_call(
        functools.partial(_k5_body, d=d),
        grid=(n // rb1,),
        in_specs=[
            pl.BlockSpec((rb1, d2), lambda i: (i, 0)),
            pl.BlockSpec((1, d2), lambda i: (0, 0)),
        ],
        out_specs=pl.BlockSpec((rb1, 2), lambda i: (i, 0)),
        out_shape=jax.ShapeDtypeStruct((n, 2), jnp.float32),
    )(zcat, w2)

    logits = jnp.concatenate([sc[:, 0], sc[:, 1]]).reshape(1, 2 * n)
    return logits
